# baseline (speedup 1.0000x reference)
import functools

import jax
import jax.numpy as jnp
from jax import lax
from jax.experimental import pallas as pl
from jax.experimental.pallas import tpu as pltpu
from jax.experimental.pallas import tpu_sc as plsc

_ALPHA = 32.0
_BLK = 256


def _router_body(x_ref, rw_ref, la_ref, lb_ref,
                 oh1_ref, oh2_ref, w1_ref, w2_ref, lora_ref):
    x = x_ref[...]
    logits = jnp.dot(x, rw_ref[...], preferred_element_type=jnp.float32)
    m = jnp.max(logits, axis=-1, keepdims=True)
    p = jnp.exp(logits - m)
    p = p / jnp.sum(p, axis=-1, keepdims=True)
    eidx = jax.lax.broadcasted_iota(jnp.int32, p.shape, 1)
    i1 = jnp.argmax(p, axis=-1)
    oh1 = (eidx == i1[:, None]).astype(jnp.float32)
    v1 = jnp.max(p, axis=-1)
    pm = jnp.where(oh1 > 0, -jnp.inf, p)
    i2 = jnp.argmax(pm, axis=-1)
    oh2 = (eidx == i2[:, None]).astype(jnp.float32)
    v2 = jnp.max(pm, axis=-1)
    den = v1 + v2
    oh1_ref[...] = oh1
    oh2_ref[...] = oh2
    w1_ref[...] = (v1 / den)[:, None]
    w2_ref[...] = (v2 / den)[:, None]
    r = la_ref.shape[1]
    lora = jnp.dot(jnp.dot(x, la_ref[...], preferred_element_type=jnp.float32),
                   lb_ref[...], preferred_element_type=jnp.float32)
    lora_ref[...] = lora * (_ALPHA / r)


def _router(x, router_w, lora_A, lora_B):
    t, h = x.shape
    e = router_w.shape[1]
    r = lora_A.shape[1]
    tb = 512
    return pl.pallas_call(
        _router_body,
        grid=(t // tb,),
        in_specs=[
            pl.BlockSpec((tb, h), lambda i: (i, 0)),
            pl.BlockSpec((h, e), lambda i: (0, 0)),
            pl.BlockSpec((h, r), lambda i: (0, 0)),
            pl.BlockSpec((r, h), lambda i: (0, 0)),
        ],
        out_specs=[
            pl.BlockSpec((tb, e), lambda i: (i, 0)),
            pl.BlockSpec((tb, e), lambda i: (i, 0)),
            pl.BlockSpec((tb, 1), lambda i: (i, 0)),
            pl.BlockSpec((tb, 1), lambda i: (i, 0)),
            pl.BlockSpec((tb, h), lambda i: (i, 0)),
        ],
        out_shape=[
            jax.ShapeDtypeStruct((t, e), jnp.float32),
            jax.ShapeDtypeStruct((t, e), jnp.float32),
            jax.ShapeDtypeStruct((t, 1), jnp.float32),
            jax.ShapeDtypeStruct((t, 1), jnp.float32),
            jax.ShapeDtypeStruct((t, h), jnp.float32),
        ],
    )(x, router_w, lora_A, lora_B)


def _dispatch_body(oh1_ref, oh2_ref, pos_ref, be_ref, rank_ref):
    t, e = oh1_ref.shape
    npairs = 2 * t
    nchunks = npairs // _BLK
    half = nchunks // 2
    nb_total = pos_ref.shape[0] // _BLK + e

    rows = jax.lax.broadcasted_iota(jnp.int32, (_BLK, _BLK), 0)
    cols = jax.lax.broadcasted_iota(jnp.int32, (_BLK, _BLK), 1)
    lts = (rows > cols).astype(jnp.float32)

    def load_chunk(j):
        o = (lax.rem(j, half)) * _BLK
        a = oh1_ref[pl.ds(o, _BLK), :]
        b = oh2_ref[pl.ds(o, _BLK), :]
        return jnp.where(j < half, a, b)

    def pass1(j, carry):
        oh = load_chunk(j)
        rank = jnp.dot(lts, oh, preferred_element_type=jnp.float32) + carry
        rank_ref[pl.ds(j * _BLK, _BLK), :] = rank
        return carry + jnp.sum(oh, axis=0, keepdims=True)

    cnt = lax.fori_loop(0, nchunks, pass1, jnp.zeros((1, e), jnp.float32))

    nb = jnp.floor((cnt + (_BLK - 1)) * (1.0 / _BLK))
    ri = jax.lax.broadcasted_iota(jnp.int32, (e, e), 0)
    ci = jax.lax.broadcasted_iota(jnp.int32, (e, e), 1)
    ut = (ri < ci).astype(jnp.float32)
    start_blk = jnp.dot(nb, ut, preferred_element_type=jnp.float32)
    off = start_blk * float(_BLK)

    def pass2(j, _):
        oh = load_chunk(j)
        rank = rank_ref[pl.ds(j * _BLK, _BLK), :]
        posf = jnp.sum(oh * (rank + off), axis=1, keepdims=True)
        pos_ref[pl.ds(j * _BLK, _BLK), :] = posf.astype(jnp.int32)
        return 0

    lax.fori_loop(0, nchunks, pass2, 0)

    bi = jax.lax.broadcasted_iota(jnp.int32, (nb_total, e), 0).astype(jnp.float32)
    be = jnp.sum((bi >= start_blk).astype(jnp.float32), axis=1, keepdims=True)
    be_ref[...] = be.astype(jnp.int32) - 1


def _dispatch(oh1, oh2, nb_total):
    t, e = oh1.shape
    return pl.pallas_call(
        _dispatch_body,
        out_shape=[
            jax.ShapeDtypeStruct((2 * t, 1), jnp.int32),
            jax.ShapeDtypeStruct((nb_total, 1), jnp.int32),
        ],
        scratch_shapes=[pltpu.VMEM((2 * t, e), jnp.float32)],
    )(oh1, oh2)


def _make_sc_scatter(t, h, cap, nw):
    rows_per_w = 2 * t // nw
    nchunk = rows_per_w // 64
    mesh = plsc.VectorSubcoreMesh(core_axis_name="c", subcore_axis_name="s",
                                  num_cores=2, num_subcores=16)

    @functools.partial(
        pl.kernel,
        out_type=(
            jax.ShapeDtypeStruct((cap, h), jnp.float32),
            jax.ShapeDtypeStruct((cap,), jnp.float32),
        ),
        mesh=mesh,
        scratch_types=[
            pltpu.VMEM((nchunk, 64), jnp.int32),
            pltpu.VMEM((nchunk, 64), jnp.float32),
            pltpu.VMEM((64, h), jnp.float32),
            pltpu.SemaphoreType.DMA,
        ],
    )
    def sc_scatter(pos2d, w2d, x, xs, wsort, idx_v, w_v, xbuf, sem):
        nc = 2
        wid = lax.axis_index("s") * nc + lax.axis_index("c")
        row0 = wid * nchunk
        pltpu.sync_copy(pos2d.at[pl.ds(row0, nchunk)], idx_v)
        pltpu.sync_copy(w2d.at[pl.ds(row0, nchunk)], w_v)
        tbase = lax.rem(wid, nw // 2) * rows_per_w
        for c in range(nchunk):
            pltpu.sync_copy(x.at[pl.ds(tbase + c * 64, 64)], xbuf)
            pltpu.async_copy(xbuf, xs.at[idx_v.at[c]], sem).wait()
            pltpu.async_copy(w_v.at[c], wsort.at[idx_v.at[c]], sem).wait()

    return sc_scatter


def _gmm_body(be_ref, xs_ref, ws_ref, wg_ref, wu_ref, wd_ref, out_ref):
    xb = xs_ref[...]
    g = jnp.dot(xb, wg_ref[0], preferred_element_type=jnp.float32)
    u = jnp.dot(xb, wu_ref[0], preferred_element_type=jnp.float32)
    hdn = (g * jax.nn.sigmoid(g)) * u * ws_ref[...]
    out_ref[...] = jnp.dot(hdn, wd_ref[0], preferred_element_type=jnp.float32)


def _gmm(xs, wsort, W_gate, W_up, W_down, be):
    cap, h = xs.shape
    e, _, esz = W_gate.shape
    nb_total = cap // _BLK
    grid_spec = pltpu.PrefetchScalarGridSpec(
        num_scalar_prefetch=1,
        grid=(nb_total,),
        in_specs=[
            pl.BlockSpec((_BLK, h), lambda i, s: (i, 0)),
            pl.BlockSpec((_BLK, 1), lambda i, s: (i, 0)),
            pl.BlockSpec((1, h, esz), lambda i, s: (s[i], 0, 0)),
            pl.BlockSpec((1, h, esz), lambda i, s: (s[i], 0, 0)),
            pl.BlockSpec((1, esz, h), lambda i, s: (s[i], 0, 0)),
        ],
        out_specs=pl.BlockSpec((_BLK, h), lambda i, s: (i, 0)),
    )
    return pl.pallas_call(
        _gmm_body,
        grid_spec=grid_spec,
        out_shape=jax.ShapeDtypeStruct((cap, h), jnp.float32),
        compiler_params=pltpu.CompilerParams(
            dimension_semantics=("arbitrary",),
        ),
    )(be, xs, wsort, W_gate, W_up, W_down)


def _make_sc_gather(t, h, cap, nw):
    pairs_per_w = 2 * t // nw
    nchunk = pairs_per_w // 64
    mesh = plsc.VectorSubcoreMesh(core_axis_name="c", subcore_axis_name="s",
                                  num_cores=2, num_subcores=16)

    @functools.partial(
        pl.kernel,
        out_type=jax.ShapeDtypeStruct((2 * t, h), jnp.float32),
        mesh=mesh,
        scratch_types=[
            pltpu.VMEM((nchunk, 64), jnp.int32),
            pltpu.VMEM((64, h), jnp.float32),
            pltpu.SemaphoreType.DMA,
        ],
    )
    def sc_gather(ys, pos2d, ytcat, idx_v, buf, sem):
        nc = 2
        wid = lax.axis_index("s") * nc + lax.axis_index("c")
        pltpu.sync_copy(pos2d.at[pl.ds(wid * nchunk, nchunk)], idx_v)
        for c in range(nchunk):
            p0 = wid * pairs_per_w + c * 64
            pltpu.async_copy(ys.at[idx_v.at[c]], buf, sem).wait()
            pltpu.sync_copy(buf, ytcat.at[pl.ds(p0, 64)])

    return sc_gather


def _finalize_body(lora_ref, y1_ref, y2_ref, out_ref):
    out_ref[...] = lora_ref[...] + y1_ref[...] + y2_ref[...]


def _finalize(lora, ytcat):
    t, h = lora.shape
    tb = 512
    return pl.pallas_call(
        _finalize_body,
        grid=(t // tb,),
        in_specs=[
            pl.BlockSpec((tb, h), lambda i: (i, 0)),
            pl.BlockSpec((tb, h), lambda i: (i, 0)),
            pl.BlockSpec((tb, h), lambda i: (i + t // tb, 0)),
        ],
        out_specs=pl.BlockSpec((tb, h), lambda i: (i, 0)),
        out_shape=jax.ShapeDtypeStruct((t, h), jnp.float32),
    )(lora, ytcat, ytcat)


def kernel(input, router_w, W_gate, W_up, W_down, lora_A, lora_B):
    b, s, h = input.shape
    t = b * s
    e = router_w.shape[1]
    x = input.reshape(t, h)
    nb_total = (2 * t) // _BLK + e
    cap = nb_total * _BLK
    nw = 32

    oh1, oh2, w1, w2, lora = _router(x, router_w, lora_A, lora_B)
    pos, be = _dispatch(oh1, oh2, nb_total)

    pos2d = pos.reshape(2 * t // 64, 64)
    w2d = jnp.concatenate([w1, w2], axis=0).reshape(2 * t // 64, 64)

    xs, wsort = _make_sc_scatter(t, h, cap, nw)(pos2d, w2d, x)
    ys = _gmm(xs, wsort.reshape(cap, 1), W_gate, W_up, W_down,
              be.reshape(nb_total))
    ytcat = _make_sc_gather(t, h, cap, nw)(ys, pos2d)
    out = _finalize(lora, ytcat)
    return out.reshape(b, s, h)
